# Initial kernel scaffold; baseline (speedup 1.0000x reference)
import functools

import jax
import jax.numpy as jnp
from jax import lax
from jax.experimental import pallas as pl
from jax.experimental.pallas import tpu as pltpu
from jax.experimental.pallas import tpu_sc as plsc

N = 10000
E = 320000
D = 128
NP = 10240
NS = 16
ET = E // NS
EG = ET // 16
TN = NP // NS
TG = TN // 16

_f32 = jnp.float32


def _sc_body(src_hbm, dst_hbm, w_hbm, v1_hbm, v2_hbm, v3_hbm,
             src_v, dst_v, w_v, norm_v, acc_v, gbuf_v, tmp2_v, red_v,
             stage_sh, glob_sh):
    cid = lax.axis_index("c")
    wid = lax.axis_index("s")
    ebase = wid * ET
    nbase = wid * TN

    pltpu.sync_copy(src_hbm.at[pl.ds(ebase, ET)], src_v)
    pltpu.sync_copy(dst_hbm.at[pl.ds(ebase, ET)], dst_v)
    pltpu.sync_copy(w_hbm.at[pl.ds(ebase, ET)], w_v)

    def _zero_acc():
        def body(j, _):
            acc_v[pl.ds(j * 16, 16)] = jnp.zeros((16,), _f32)
            return 0
        lax.fori_loop(0, NP // 16, body, 0)

    def _publish_and_reduce():
        pltpu.sync_copy(acc_v, stage_sh.at[wid])
        plsc.subcore_barrier()
        pltpu.sync_copy(stage_sh.at[:, pl.ds(nbase, TN)], tmp2_v)

        def body(j, _):
            s = jnp.zeros((16,), _f32)
            for t in range(NS):
                s = s + tmp2_v[t, pl.ds(j * 16, 16)]
            red_v[pl.ds(j * 16, 16)] = s
            return 0
        lax.fori_loop(0, TG, body, 0)

    _zero_acc()

    def deg_body(g, _):
        d16 = dst_v[pl.ds(g * 16, 16)]
        w16 = w_v[pl.ds(g * 16, 16)]
        plsc.addupdate_scatter(acc_v, [d16], w16)
        return 0
    lax.fori_loop(0, EG, deg_body, 0)
    _publish_and_reduce()

    def dinv_body(j, _):
        xv = red_v[pl.ds(j * 16, 16)]
        nz = xv > 0.0
        xs = jnp.where(nz, xv, 1.0)
        ibits = plsc.bitcast(xs, jnp.int32)
        ibits = jnp.int32(0x5F3759DF) - lax.shift_right_logical(ibits, 1)
        y = plsc.bitcast(ibits, _f32)
        hx = xs * 0.5
        y = y * (1.5 - hx * y * y)
        y = y * (1.5 - hx * y * y)
        y = y * (1.5 - hx * y * y)
        red_v[pl.ds(j * 16, 16)] = jnp.where(nz, y, 0.0)
        return 0
    lax.fori_loop(0, TG, dinv_body, 0)
    pltpu.sync_copy(red_v, glob_sh.at[pl.ds(nbase, TN)])
    plsc.subcore_barrier()
    pltpu.sync_copy(glob_sh, gbuf_v)

    def norm_body(g, _):
        s16 = src_v[pl.ds(g * 16, 16)]
        d16 = dst_v[pl.ds(g * 16, 16)]
        a = plsc.load_gather(gbuf_v, [s16])
        b = plsc.load_gather(gbuf_v, [d16])
        norm_v[pl.ds(g * 16, 16)] = a * w_v[pl.ds(g * 16, 16)] * b
        return 0
    lax.fori_loop(0, EG, norm_body, 0)

    def fill_ones(j, _):
        gbuf_v[pl.ds(j * 16, 16)] = jnp.ones((16,), _f32)
        return 0
    lax.fori_loop(0, NP // 16, fill_ones, 0)

    for r, out_hbm in enumerate((v1_hbm, v2_hbm, v3_hbm)):
        _zero_acc()

        def spmv_body(g, _):
            s16 = src_v[pl.ds(g * 16, 16)]
            d16 = dst_v[pl.ds(g * 16, 16)]
            vk = plsc.load_gather(gbuf_v, [d16])
            plsc.addupdate_scatter(acc_v, [s16],
                                   norm_v[pl.ds(g * 16, 16)] * vk)
            return 0
        lax.fori_loop(0, EG, spmv_body, 0)
        _publish_and_reduce()

        @pl.when(cid == 0)
        def _():
            pltpu.sync_copy(red_v, out_hbm.at[pl.ds(nbase, TN)])
        if r < 2:
            pltpu.sync_copy(red_v, glob_sh.at[pl.ds(nbase, TN)])
            plsc.subcore_barrier()
            pltpu.sync_copy(glob_sh, gbuf_v)


def _sc_spmv(src, dst, w):
    mesh = plsc.VectorSubcoreMesh(core_axis_name="c", subcore_axis_name="s")
    f = pl.kernel(
        _sc_body,
        out_type=(jax.ShapeDtypeStruct((NP,), _f32),) * 3,
        mesh=mesh,
        scratch_types=[
            pltpu.VMEM((ET,), jnp.int32),
            pltpu.VMEM((ET,), jnp.int32),
            pltpu.VMEM((ET,), _f32),
            pltpu.VMEM((ET,), _f32),
            pltpu.VMEM((NP,), _f32),
            pltpu.VMEM((NP,), _f32),
            pltpu.VMEM((NS, TN), _f32),
            pltpu.VMEM((TN,), _f32),
            pltpu.VMEM_SHARED((NS, NP), _f32),
            pltpu.VMEM_SHARED((NP,), _f32),
        ],
    )
    return f(src, dst, w)


def _tc1_body(x_ref, w1_ref, w2_ref, w3_ref, wl_ref, y_ref):
    wl0 = wl_ref[...][:, 0]
    u3 = jnp.sum(w3_ref[...] * wl0[None, :], axis=1)
    u2 = jnp.sum(w2_ref[...] * u3[None, :], axis=1)
    u1 = jnp.sum(w1_ref[...] * u2[None, :], axis=1)
    y_ref[...] = jnp.sum(x_ref[...] * u1[None, :], axis=1, keepdims=True)


def _tc2_body(v1_ref, v2_ref, v3_ref, y_ref, w2_ref, w3_ref, wl_ref,
              b1_ref, b2_ref, b3_ref, bl_ref, o_ref):
    wl0 = wl_ref[...][:, 0]
    u3 = jnp.sum(w3_ref[...] * wl0[None, :], axis=1)
    u2 = jnp.sum(w2_ref[...] * u3[None, :], axis=1)
    t1 = jnp.sum(v3_ref[...] * y_ref[...])
    t2 = jnp.sum(v2_ref[...]) * jnp.sum(b1_ref[...][0, :] * u2)
    t3 = jnp.sum(v1_ref[...]) * jnp.sum(b2_ref[...][0, :] * u3)
    t4 = jnp.float32(N) * jnp.sum(b3_ref[...][0, :] * wl0)
    o_ref[...] = jnp.reshape(t1 + t2 + t3 + t4 + bl_ref[...][0, 0], (1, 1))


def kernel(x, edge_index, edge_attr, W1, b1, W2, b2, W3, b3, Wl, bl):
    src = edge_index[0]
    dst = edge_index[1]
    x_pad = jnp.pad(x, ((0, NP - N), (0, 0)))

    y = pl.pallas_call(
        _tc1_body,
        out_shape=jax.ShapeDtypeStruct((NP, 1), _f32),
    )(x_pad, W1, W2, W3, Wl)

    v1, v2, v3 = _sc_spmv(src, dst, edge_attr)

    out = pl.pallas_call(
        _tc2_body,
        out_shape=jax.ShapeDtypeStruct((1, 1), _f32),
    )(v1.reshape(NP // 128, 128), v2.reshape(NP // 128, 128),
      v3.reshape(NP // 128, 128), y.reshape(NP // 128, 128),
      W2, W3, Wl, b1.reshape(1, 16), b2.reshape(1, 16), b3.reshape(1, 16),
      bl.reshape(1, 1))
    return out


# speedup vs baseline: 80.0549x; 80.0549x over previous
import functools

import jax
import jax.numpy as jnp
from jax import lax
from jax.experimental import pallas as pl
from jax.experimental.pallas import tpu as pltpu
from jax.experimental.pallas import tpu_sc as plsc

N = 10000
E = 320000
D = 128
NP = 10240
NS = 16
ET = E // NS
EG = ET // 16
TN = NP // NS
TG = TN // 16

_f32 = jnp.float32


def _sc_body(src_hbm, dst_hbm, w_hbm, v1_hbm, v2_hbm, v3_hbm,
             src_v, dst_v, w_v, norm_v, acc_v, gbuf_v, tmp2_v, red_v,
             stage_sh, glob_sh):
    cid = lax.axis_index("c")
    wid = lax.axis_index("s")
    ebase = wid * ET
    nbase = wid * TN

    pltpu.sync_copy(src_hbm.at[pl.ds(ebase, ET)], src_v)
    pltpu.sync_copy(dst_hbm.at[pl.ds(ebase, ET)], dst_v)
    pltpu.sync_copy(w_hbm.at[pl.ds(ebase, ET)], w_v)

    def _zero_acc():
        def body(j, _):
            acc_v[pl.ds(j * 16, 16)] = jnp.zeros((16,), _f32)
            return 0
        lax.fori_loop(0, NP // 16, body, 0)

    def _publish_and_reduce():
        pltpu.sync_copy(acc_v, stage_sh.at[wid])
        plsc.subcore_barrier()
        pltpu.sync_copy(stage_sh.at[:, pl.ds(nbase, TN)], tmp2_v)

        def body(j, _):
            s = jnp.zeros((16,), _f32)
            for t in range(NS):
                s = s + tmp2_v[t, pl.ds(j * 16, 16)]
            red_v[pl.ds(j * 16, 16)] = s
            return 0
        lax.fori_loop(0, TG, body, 0)

    _zero_acc()

    def deg_body(g, _):
        d16 = dst_v[pl.ds(g * 16, 16)]
        w16 = w_v[pl.ds(g * 16, 16)]
        plsc.addupdate_scatter(acc_v, [d16], w16)
        return 0
    lax.fori_loop(0, EG, deg_body, 0)
    _publish_and_reduce()

    def dinv_body(j, _):
        xv = red_v[pl.ds(j * 16, 16)]
        nz = xv > 0.0
        xs = jnp.where(nz, xv, 1.0)
        ibits = plsc.bitcast(xs, jnp.int32)
        ibits = jnp.int32(0x5F3759DF) - lax.shift_right_logical(ibits, 1)
        y = plsc.bitcast(ibits, _f32)
        hx = xs * 0.5
        y = y * (1.5 - hx * y * y)
        y = y * (1.5 - hx * y * y)
        y = y * (1.5 - hx * y * y)
        red_v[pl.ds(j * 16, 16)] = jnp.where(nz, y, 0.0)
        return 0
    lax.fori_loop(0, TG, dinv_body, 0)
    pltpu.sync_copy(red_v, glob_sh.at[pl.ds(nbase, TN)])
    plsc.subcore_barrier()
    pltpu.sync_copy(glob_sh, gbuf_v)

    def norm_body(g, _):
        s16 = src_v[pl.ds(g * 16, 16)]
        d16 = dst_v[pl.ds(g * 16, 16)]
        a = plsc.load_gather(gbuf_v, [s16])
        b = plsc.load_gather(gbuf_v, [d16])
        norm_v[pl.ds(g * 16, 16)] = a * w_v[pl.ds(g * 16, 16)] * b
        return 0
    lax.fori_loop(0, EG, norm_body, 0)

    def fill_ones(j, _):
        gbuf_v[pl.ds(j * 16, 16)] = jnp.ones((16,), _f32)
        return 0
    lax.fori_loop(0, NP // 16, fill_ones, 0)

    for r, out_hbm in enumerate((v1_hbm, v2_hbm, v3_hbm)):
        _zero_acc()

        def spmv_body(g, _):
            s16 = src_v[pl.ds(g * 16, 16)]
            d16 = dst_v[pl.ds(g * 16, 16)]
            vk = plsc.load_gather(gbuf_v, [d16])
            plsc.addupdate_scatter(acc_v, [s16],
                                   norm_v[pl.ds(g * 16, 16)] * vk)
            return 0
        lax.fori_loop(0, EG, spmv_body, 0)
        _publish_and_reduce()

        @pl.when(cid == 0)
        def _():
            pltpu.sync_copy(red_v, out_hbm.at[pl.ds(nbase, TN)])
        if r < 2:
            pltpu.sync_copy(red_v, glob_sh.at[pl.ds(nbase, TN)])
            plsc.subcore_barrier()
            pltpu.sync_copy(glob_sh, gbuf_v)


def _sc_spmv(src, dst, w):
    mesh = plsc.VectorSubcoreMesh(core_axis_name="c", subcore_axis_name="s")
    f = pl.kernel(
        _sc_body,
        out_type=(jax.ShapeDtypeStruct((NP,), _f32),) * 3,
        mesh=mesh,
        scratch_types=[
            pltpu.VMEM((ET,), jnp.int32),
            pltpu.VMEM((ET,), jnp.int32),
            pltpu.VMEM((ET,), _f32),
            pltpu.VMEM((ET,), _f32),
            pltpu.VMEM((NP,), _f32),
            pltpu.VMEM((NP,), _f32),
            pltpu.VMEM((NS, TN), _f32),
            pltpu.VMEM((TN,), _f32),
            pltpu.VMEM_SHARED((NS, NP), _f32),
            pltpu.VMEM_SHARED((NP,), _f32),
        ],
        compiler_params=pltpu.CompilerParams(needs_layout_passes=False),
    )
    return f(src, dst, w)


def _tc1_body(x_ref, w1_ref, w2_ref, w3_ref, wl_ref, y_ref):
    wl0 = wl_ref[...][:, 0]
    u3 = jnp.sum(w3_ref[...] * wl0[None, :], axis=1)
    u2 = jnp.sum(w2_ref[...] * u3[None, :], axis=1)
    u1 = jnp.sum(w1_ref[...] * u2[None, :], axis=1)
    y_ref[...] = jnp.sum(x_ref[...] * u1[None, :], axis=1, keepdims=True)


def _tc2_body(v1_ref, v2_ref, v3_ref, y_ref, w2_ref, w3_ref, wl_ref,
              b1_ref, b2_ref, b3_ref, bl_ref, o_ref):
    wl0 = wl_ref[...][:, 0]
    u3 = jnp.sum(w3_ref[...] * wl0[None, :], axis=1)
    u2 = jnp.sum(w2_ref[...] * u3[None, :], axis=1)
    t1 = jnp.sum(v3_ref[...] * y_ref[...])
    t2 = jnp.sum(v2_ref[...]) * jnp.sum(b1_ref[...][0, :] * u2)
    t3 = jnp.sum(v1_ref[...]) * jnp.sum(b2_ref[...][0, :] * u3)
    t4 = jnp.float32(N) * jnp.sum(b3_ref[...][0, :] * wl0)
    o_ref[...] = jnp.reshape(t1 + t2 + t3 + t4 + bl_ref[...][0, 0], (1, 1))


def kernel(x, edge_index, edge_attr, W1, b1, W2, b2, W3, b3, Wl, bl):
    src = edge_index[0]
    dst = edge_index[1]
    x_pad = jnp.pad(x, ((0, NP - N), (0, 0)))

    y = pl.pallas_call(
        _tc1_body,
        out_shape=jax.ShapeDtypeStruct((NP, 1), _f32),
    )(x_pad, W1, W2, W3, Wl)

    v1, v2, v3 = _sc_spmv(src, dst, edge_attr)

    out = pl.pallas_call(
        _tc2_body,
        out_shape=jax.ShapeDtypeStruct((1, 1), _f32),
    )(v1.reshape(NP // 128, 128), v2.reshape(NP // 128, 128),
      v3.reshape(NP // 128, 128), y.reshape(NP // 128, 128),
      W2, W3, Wl, b1.reshape(1, 16), b2.reshape(1, 16), b3.reshape(1, 16),
      bl.reshape(1, 1))
    return out


# speedup vs baseline: 138.3395x; 1.7281x over previous
import functools

import jax
import jax.numpy as jnp
from jax import lax
from jax.experimental import pallas as pl
from jax.experimental.pallas import tpu as pltpu
from jax.experimental.pallas import tpu_sc as plsc

N = 10000
E = 320000
D = 128
NP = 10240
NS = 16
ET = E // NS
EG = ET // 16
TN = NP // NS
TG = TN // 16

_f32 = jnp.float32


def _sc_body(src_hbm, dst_hbm, w_hbm, v1_hbm, v2_hbm, v3_hbm,
             src_v, dst_v, w_v, norm_v, acc_v, gbuf_v, tmp2_v, red_v,
             stage_sh, glob_sh):
    cid = lax.axis_index("c")
    wid = lax.axis_index("s")
    ebase = wid * ET
    nbase = wid * TN

    pltpu.sync_copy(src_hbm.at[pl.ds(ebase, ET)], src_v)
    pltpu.sync_copy(dst_hbm.at[pl.ds(ebase, ET)], dst_v)
    pltpu.sync_copy(w_hbm.at[pl.ds(ebase, ET)], w_v)

    def _zero_acc():
        @plsc.parallel_loop(0, NP, step=16, unroll=8)
        def _(off):
            acc_v[pl.ds(off, 16)] = jnp.zeros((16,), _f32)

    def _publish_and_reduce():
        pltpu.sync_copy(acc_v, stage_sh.at[wid])
        plsc.subcore_barrier()
        pltpu.sync_copy(stage_sh.at[:, pl.ds(nbase, TN)], tmp2_v)

        def body(j, _):
            s = jnp.zeros((16,), _f32)
            for t in range(NS):
                s = s + tmp2_v[t, pl.ds(j * 16, 16)]
            red_v[pl.ds(j * 16, 16)] = s
            return 0
        lax.fori_loop(0, TG, body, 0)

    _zero_acc()

    @plsc.parallel_loop(0, ET, step=16, unroll=4)
    def _(off):
        d16 = dst_v[pl.ds(off, 16)]
        w16 = w_v[pl.ds(off, 16)]
        plsc.addupdate_scatter(acc_v, [d16], w16)
    _publish_and_reduce()

    def dinv_body(j, _):
        xv = red_v[pl.ds(j * 16, 16)]
        nz = xv > 0.0
        xs = jnp.where(nz, xv, 1.0)
        ibits = plsc.bitcast(xs, jnp.int32)
        ibits = jnp.int32(0x5F3759DF) - lax.shift_right_logical(ibits, 1)
        y = plsc.bitcast(ibits, _f32)
        hx = xs * 0.5
        y = y * (1.5 - hx * y * y)
        y = y * (1.5 - hx * y * y)
        y = y * (1.5 - hx * y * y)
        red_v[pl.ds(j * 16, 16)] = jnp.where(nz, y, 0.0)
        return 0
    lax.fori_loop(0, TG, dinv_body, 0)
    pltpu.sync_copy(red_v, glob_sh.at[pl.ds(nbase, TN)])
    plsc.subcore_barrier()
    pltpu.sync_copy(glob_sh, gbuf_v)

    for r, out_hbm in enumerate((v1_hbm, v2_hbm, v3_hbm)):
        _zero_acc()

        if r == 0:
            @plsc.parallel_loop(0, ET, step=16, unroll=4)
            def _(off):
                s16 = src_v[pl.ds(off, 16)]
                d16 = dst_v[pl.ds(off, 16)]
                a = plsc.load_gather(gbuf_v, [s16])
                b = plsc.load_gather(gbuf_v, [d16])
                nv = a * w_v[pl.ds(off, 16)] * b
                norm_v[pl.ds(off, 16)] = nv
                plsc.addupdate_scatter(acc_v, [s16], nv)
        else:
            @plsc.parallel_loop(0, ET, step=16, unroll=4)
            def _(off):
                s16 = src_v[pl.ds(off, 16)]
                d16 = dst_v[pl.ds(off, 16)]
                vk = plsc.load_gather(gbuf_v, [d16])
                plsc.addupdate_scatter(acc_v, [s16],
                                       norm_v[pl.ds(off, 16)] * vk)
        _publish_and_reduce()

        @pl.when(cid == 0)
        def _():
            pltpu.sync_copy(red_v, out_hbm.at[pl.ds(nbase, TN)])
        if r < 2:
            pltpu.sync_copy(red_v, glob_sh.at[pl.ds(nbase, TN)])
            plsc.subcore_barrier()
            pltpu.sync_copy(glob_sh, gbuf_v)


def _sc_spmv(src, dst, w):
    mesh = plsc.VectorSubcoreMesh(core_axis_name="c", subcore_axis_name="s")
    f = pl.kernel(
        _sc_body,
        out_type=(jax.ShapeDtypeStruct((NP,), _f32),) * 3,
        mesh=mesh,
        scratch_types=[
            pltpu.VMEM((ET,), jnp.int32),
            pltpu.VMEM((ET,), jnp.int32),
            pltpu.VMEM((ET,), _f32),
            pltpu.VMEM((ET,), _f32),
            pltpu.VMEM((NP,), _f32),
            pltpu.VMEM((NP,), _f32),
            pltpu.VMEM((NS, TN), _f32),
            pltpu.VMEM((TN,), _f32),
            pltpu.VMEM_SHARED((NS, NP), _f32),
            pltpu.VMEM_SHARED((NP,), _f32),
        ],
        compiler_params=pltpu.CompilerParams(needs_layout_passes=False),
    )
    return f(src, dst, w)


def _tc1_body(x_ref, w1_ref, w2_ref, w3_ref, wl_ref, y_ref):
    wl0 = wl_ref[...][:, 0]
    u3 = jnp.sum(w3_ref[...] * wl0[None, :], axis=1)
    u2 = jnp.sum(w2_ref[...] * u3[None, :], axis=1)
    u1 = jnp.sum(w1_ref[...] * u2[None, :], axis=1)
    y_ref[...] = jnp.sum(x_ref[...] * u1[None, :], axis=1, keepdims=True)


def _tc2_body(v1_ref, v2_ref, v3_ref, y_ref, w2_ref, w3_ref, wl_ref,
              b1_ref, b2_ref, b3_ref, bl_ref, o_ref):
    wl0 = wl_ref[...][:, 0]
    u3 = jnp.sum(w3_ref[...] * wl0[None, :], axis=1)
    u2 = jnp.sum(w2_ref[...] * u3[None, :], axis=1)
    t1 = jnp.sum(v3_ref[...] * y_ref[...])
    t2 = jnp.sum(v2_ref[...]) * jnp.sum(b1_ref[...][0, :] * u2)
    t3 = jnp.sum(v1_ref[...]) * jnp.sum(b2_ref[...][0, :] * u3)
    t4 = jnp.float32(N) * jnp.sum(b3_ref[...][0, :] * wl0)
    o_ref[...] = jnp.reshape(t1 + t2 + t3 + t4 + bl_ref[...][0, 0], (1, 1))


def kernel(x, edge_index, edge_attr, W1, b1, W2, b2, W3, b3, Wl, bl):
    src = edge_index[0]
    dst = edge_index[1]
    x_pad = jnp.pad(x, ((0, NP - N), (0, 0)))

    y = pl.pallas_call(
        _tc1_body,
        out_shape=jax.ShapeDtypeStruct((NP, 1), _f32),
    )(x_pad, W1, W2, W3, Wl)

    v1, v2, v3 = _sc_spmv(src, dst, edge_attr)

    out = pl.pallas_call(
        _tc2_body,
        out_shape=jax.ShapeDtypeStruct((1, 1), _f32),
    )(v1.reshape(NP // 128, 128), v2.reshape(NP // 128, 128),
      v3.reshape(NP // 128, 128), y.reshape(NP // 128, 128),
      W2, W3, Wl, b1.reshape(1, 16), b2.reshape(1, 16), b3.reshape(1, 16),
      bl.reshape(1, 1))
    return out
